# baseline (speedup 1.0000x reference)
import functools

import jax
import jax.numpy as jnp
from jax import lax
from jax.experimental import pallas as pl


def _conv(x, w, stride=1, pad=0):
    return lax.conv_general_dilated(
        x, w, (stride, stride), [(pad, pad), (pad, pad)],
        dimension_numbers=('NCHW', 'OIHW', 'NCHW'))


def _bn(x, p):
    return (x - p['m'][None, :, None, None]) / jnp.sqrt(
        p['v'][None, :, None, None] + 1e-5) * p['g'][None, :, None, None] \
        + p['b'][None, :, None, None]


def _bottleneck(x, blk, s):
    out = jax.nn.relu(_bn(_conv(x, blk['w1']), blk['bn1']))
    out = jax.nn.relu(_bn(_conv(out, blk['w2'], s, 1), blk['bn2']))
    out = _bn(_conv(out, blk['w3']), blk['bn3'])
    out = out + (jnp.asarray(blk['stride']) - s).astype(out.dtype)
    if 'wd' in blk:
        idn = _bn(_conv(x, blk['wd'], s), blk['bnd'])
    else:
        idn = x
    return jax.nn.relu(out + idn)


def _encode(x, enc):
    x = _conv(x, enc['conv1'], 2, 3)
    x = jax.nn.relu(_bn(x, enc['bn1']))
    x = lax.reduce_window(x, -jnp.inf, lax.max, (1, 1, 3, 3), (1, 1, 2, 2),
                          [(0, 0), (0, 0), (1, 1), (1, 1)])
    for blk in enc['layer1']:
        x = _bottleneck(x, blk, 1)
    for i, blk in enumerate(enc['layer2']):
        x = _bottleneck(x, blk, 2 if i == 0 else 1)
    for i, blk in enumerate(enc['layer3']):
        x = _bottleneck(x, blk, 2 if i == 0 else 1)
    return jnp.mean(x, axis=(2, 3))


def _dkvb_body(e0_ref, e1_ref, cb_ref, vals_ref,
               w1e_ref, w1o_ref, b1_ref, w2_ref, b2_ref, w3_ref, b3_ref,
               out_ref):
    e0 = e0_ref[...]
    e1 = e1_ref[...]
    c00 = cb_ref[0:1, :]
    c01 = cb_ref[1:2, :]
    c10 = cb_ref[2:3, :]
    c11 = cb_ref[3:4, :]
    v00 = vals_ref[0:1, :]
    v01 = vals_ref[1:2, :]
    v10 = vals_ref[2:3, :]
    v11 = vals_ref[3:4, :]
    t0 = e0 - c00
    t1 = e1 - c01
    d0 = t0 * t0 + t1 * t1
    t0 = e0 - c10
    t1 = e1 - c11
    d1 = t0 * t0 + t1 * t1
    pick = d1 < d0
    m0 = jnp.where(pick, v10, v00)
    m1 = jnp.where(pick, v11, v01)
    h = (jnp.dot(m0, w1e_ref[...], preferred_element_type=jnp.float32)
         + jnp.dot(m1, w1o_ref[...], preferred_element_type=jnp.float32)
         + b1_ref[...])
    h = jnp.dot(h, w2_ref[...], preferred_element_type=jnp.float32) + b2_ref[...]
    h = jnp.dot(h, w3_ref[...], preferred_element_type=jnp.float32) + b3_ref[...]
    h = h - jnp.max(h, axis=1, keepdims=True)
    eh = jnp.exp(h)
    out_ref[...] = eh / jnp.sum(eh, axis=1, keepdims=True)


def _dkvb_tc(emb, codebooks, values, W1, b1, W2, b2, W3, b3):
    B = emb.shape[0]
    H = codebooks.shape[0]
    C = W3.shape[0]
    CP = 128
    e0 = emb[:, 0::2]
    e1 = emb[:, 1::2]
    cb = jnp.transpose(codebooks.reshape(H, 4), (1, 0))
    vals = jnp.transpose(values.reshape(H, 4), (1, 0))
    w1e = jnp.transpose(W1[:, 0::2], (1, 0))
    w1o = jnp.transpose(W1[:, 1::2], (1, 0))
    w2 = jnp.transpose(W2, (1, 0))
    w3 = jnp.zeros((W3.shape[1], CP), W3.dtype).at[:, :C].set(
        jnp.transpose(W3, (1, 0)))
    b3p = jnp.full((CP,), -1e30, b3.dtype).at[:C].set(b3)
    out = pl.pallas_call(
        _dkvb_body,
        out_shape=jax.ShapeDtypeStruct((B, CP), jnp.float32),
    )(e0, e1, cb, vals, w1e, w1o, b1.reshape(1, -1), w2, b2.reshape(1, -1),
      w3, b3p.reshape(1, -1))
    return out[:, :C]


def kernel(input, enc, codebooks, values, W1, b1, W2, b2, W3, b3):
    emb = lax.stop_gradient(_encode(input, enc))
    return _dkvb_tc(emb, codebooks, values, W1, b1, W2, b2, W3, b3)


# speedup vs baseline: 1.0896x; 1.0896x over previous
import functools

import jax
import jax.numpy as jnp
from jax import lax
from jax.experimental import pallas as pl


def _conv(x, w, stride=1, pad=0):
    return lax.conv_general_dilated(
        x, w, (stride, stride), [(pad, pad), (pad, pad)],
        dimension_numbers=('NCHW', 'OIHW', 'NCHW'))


def _bn(x, p):
    return (x - p['m'][None, :, None, None]) / jnp.sqrt(
        p['v'][None, :, None, None] + 1e-5) * p['g'][None, :, None, None] \
        + p['b'][None, :, None, None]


def _bottleneck(x, blk, s):
    out = jax.nn.relu(_bn(_conv(x, blk['w1']), blk['bn1']))
    out = jax.nn.relu(_bn(_conv(out, blk['w2'], s, 1), blk['bn2']))
    out = _bn(_conv(out, blk['w3']), blk['bn3'])
    out = out + (jnp.asarray(blk['stride']) - s).astype(out.dtype)
    if 'wd' in blk:
        idn = _bn(_conv(x, blk['wd'], s), blk['bnd'])
    else:
        idn = x
    return jax.nn.relu(out + idn)


def _encode(x, enc):
    x = _conv(x, enc['conv1'], 2, 3)
    x = jax.nn.relu(_bn(x, enc['bn1']))
    x = lax.reduce_window(x, -jnp.inf, lax.max, (1, 1, 3, 3), (1, 1, 2, 2),
                          [(0, 0), (0, 0), (1, 1), (1, 1)])
    for blk in enc['layer1']:
        x = _bottleneck(x, blk, 1)
    for i, blk in enumerate(enc['layer2']):
        x = _bottleneck(x, blk, 2 if i == 0 else 1)
    for i, blk in enumerate(enc['layer3']):
        x = _bottleneck(x, blk, 2 if i == 0 else 1)
    return jnp.mean(x, axis=(2, 3))


def _dot_t(x, w):
    return lax.dot_general(x, w, (((1,), (1,)), ((), ())),
                           preferred_element_type=jnp.float32)


def _dkvb_body(emb_ref, c0_ref, c1_ref, v0_ref, v1_ref,
               w1_ref, b1_ref, w2_ref, b2_ref, w3_ref, b3_ref,
               out_ref):
    emb = emb_ref[...]
    D = emb.shape[1]
    H = D // 2
    r0 = emb - c0_ref[...]
    r1 = emb - c1_ref[...]
    rows = lax.broadcasted_iota(jnp.int32, (D, H), 0)
    cols = lax.broadcasted_iota(jnp.int32, (D, H), 1)
    pair = (rows // 2 == cols).astype(jnp.float32)
    d0 = jnp.dot(r0 * r0, pair, preferred_element_type=jnp.float32)
    d1 = jnp.dot(r1 * r1, pair, preferred_element_type=jnp.float32)
    pick = (d1 < d0).astype(jnp.float32)
    pickx = _dot_t(pick, pair)
    mem = jnp.where(pickx > 0.5, v1_ref[...], v0_ref[...])
    h = _dot_t(mem, w1_ref[...]) + b1_ref[...]
    h = _dot_t(h, w2_ref[...]) + b2_ref[...]
    h = _dot_t(h, w3_ref[...]) + b3_ref[...]
    h = h - jnp.max(h, axis=1, keepdims=True)
    eh = jnp.exp(h)
    out_ref[...] = eh / jnp.sum(eh, axis=1, keepdims=True)


def _dkvb_tc(emb, codebooks, values, W1, b1, W2, b2, W3, b3):
    B, D = emb.shape
    C = W3.shape[0]
    CP = 128
    c0 = codebooks[:, 0, :].reshape(1, D)
    c1 = codebooks[:, 1, :].reshape(1, D)
    v0 = values[:, 0, :].reshape(1, D)
    v1 = values[:, 1, :].reshape(1, D)
    w3 = jnp.zeros((CP, W3.shape[1]), W3.dtype).at[:C, :].set(W3)
    b3p = jnp.full((CP,), -1e30, b3.dtype).at[:C].set(b3)
    out = pl.pallas_call(
        _dkvb_body,
        out_shape=jax.ShapeDtypeStruct((B, CP), jnp.float32),
    )(emb, c0, c1, v0, v1, W1, b1.reshape(1, -1), W2, b2.reshape(1, -1),
      w3, b3p.reshape(1, -1))
    return out[:, :C]


def kernel(input, enc, codebooks, values, W1, b1, W2, b2, W3, b3):
    emb = lax.stop_gradient(_encode(input, enc))
    return _dkvb_tc(emb, codebooks, values, W1, b1, W2, b2, W3, b3)


# speedup vs baseline: 1.0950x; 1.0050x over previous
import functools

import jax
import jax.numpy as jnp
from jax import lax
from jax.experimental import pallas as pl


def _conv(x, w, stride=1, pad=0):
    return lax.conv_general_dilated(
        x, w, (stride, stride), [(pad, pad), (pad, pad)],
        dimension_numbers=('NCHW', 'OIHW', 'NCHW'))


def _bn(x, p):
    return (x - p['m'][None, :, None, None]) / jnp.sqrt(
        p['v'][None, :, None, None] + 1e-5) * p['g'][None, :, None, None] \
        + p['b'][None, :, None, None]


def _bottleneck(x, blk, s):
    out = jax.nn.relu(_bn(_conv(x, blk['w1']), blk['bn1']))
    out = jax.nn.relu(_bn(_conv(out, blk['w2'], s, 1), blk['bn2']))
    out = _bn(_conv(out, blk['w3']), blk['bn3'])
    out = out + (jnp.asarray(blk['stride']) - s).astype(out.dtype)
    if 'wd' in blk:
        idn = _bn(_conv(x, blk['wd'], s), blk['bnd'])
    else:
        idn = x
    return jax.nn.relu(out + idn)


def _encode(x, enc):
    x = _conv(x, enc['conv1'], 2, 3)
    x = jax.nn.relu(_bn(x, enc['bn1']))
    x = lax.reduce_window(x, -jnp.inf, lax.max, (1, 1, 3, 3), (1, 1, 2, 2),
                          [(0, 0), (0, 0), (1, 1), (1, 1)])
    for blk in enc['layer1']:
        x = _bottleneck(x, blk, 1)
    for i, blk in enumerate(enc['layer2']):
        x = _bottleneck(x, blk, 2 if i == 0 else 1)
    for i, blk in enumerate(enc['layer3']):
        x = _bottleneck(x, blk, 2 if i == 0 else 1)
    return jnp.mean(x, axis=(2, 3))


def _dot_t(x, w):
    return lax.dot_general(x, w, (((1,), (1,)), ((), ())),
                           preferred_element_type=jnp.float32)


def _dkvb_body(emb_ref, c0_ref, c1_ref, v0_ref, v1_ref,
               w1_ref, b1_ref, w2_ref, b2_ref, w3_ref, b3_ref,
               out_ref):
    emb = emb_ref[...]
    D = emb.shape[1]
    H = D // 2
    r0 = emb - c0_ref[...]
    r1 = emb - c1_ref[...]
    rows = lax.broadcasted_iota(jnp.int32, (D, H), 0)
    cols = lax.broadcasted_iota(jnp.int32, (D, H), 1)
    pair = (rows // 2 == cols).astype(jnp.float32)
    d0 = jnp.dot(r0 * r0, pair, preferred_element_type=jnp.float32,
                 precision=lax.Precision.HIGHEST)
    d1 = jnp.dot(r1 * r1, pair, preferred_element_type=jnp.float32,
                 precision=lax.Precision.HIGHEST)
    pick = (d1 < d0).astype(jnp.float32)
    pickx = _dot_t(pick, pair)
    mem = jnp.where(pickx > 0.5, v1_ref[...], v0_ref[...])
    h = _dot_t(mem, w1_ref[...]) + b1_ref[...]
    h = _dot_t(h, w2_ref[...]) + b2_ref[...]
    h = _dot_t(h, w3_ref[...]) + b3_ref[...]
    h = h - jnp.max(h, axis=1, keepdims=True)
    eh = jnp.exp(h)
    out_ref[...] = eh / jnp.sum(eh, axis=1, keepdims=True)


def _dkvb_tc(emb, codebooks, values, W1, b1, W2, b2, W3, b3):
    B, D = emb.shape
    C = W3.shape[0]
    CP = 128
    c0 = codebooks[:, 0, :].reshape(1, D)
    c1 = codebooks[:, 1, :].reshape(1, D)
    v0 = values[:, 0, :].reshape(1, D)
    v1 = values[:, 1, :].reshape(1, D)
    w3 = jnp.zeros((CP, W3.shape[1]), W3.dtype).at[:C, :].set(W3)
    b3p = jnp.full((CP,), -1e30, b3.dtype).at[:C].set(b3)
    out = pl.pallas_call(
        _dkvb_body,
        out_shape=jax.ShapeDtypeStruct((B, CP), jnp.float32),
    )(emb, c0, c1, v0, v1, W1, b1.reshape(1, -1), W2, b2.reshape(1, -1),
      w3, b3p.reshape(1, -1))
    return out[:, :C]


def kernel(input, enc, codebooks, values, W1, b1, W2, b2, W3, b3):
    emb = lax.stop_gradient(_encode(input, enc))
    return _dkvb_tc(emb, codebooks, values, W1, b1, W2, b2, W3, b3)
